# baseline (speedup 1.0000x reference)
import functools

import jax
import jax.numpy as jnp
from jax import lax
from jax.experimental import pallas as pl
from jax.experimental.pallas import tpu as pltpu
from jax.experimental.pallas import tpu_sc as plsc

_N = 10000
_E = 320000
_NC = 2
_NS = 16
_NW = _NC * _NS
_B = 128
_G = 8
_K = _G * (-(-_E // (_NW * _B * _G)))
_EP = _NW * _K * _B
_NP = 10112
_RPW = _NP // _NS
_DEGW = 16

_mesh = plsc.VectorSubcoreMesh(core_axis_name="c", subcore_axis_name="s")


def _make_edge_pass(dcols):
  @functools.partial(
      pl.kernel,
      out_type=jax.ShapeDtypeStruct((_NC, _NP, dcols), jnp.float32),
      mesh=_mesh,
      scratch_types=[
          pltpu.VMEM((_K, _B), jnp.int32),
          pltpu.VMEM((_K, _B), jnp.int32),
          pltpu.VMEM((_G, _B, dcols), jnp.float32),
          pltpu.VMEM((_RPW, dcols), jnp.float32),
          pltpu.VMEM_SHARED((_NP, dcols), jnp.float32),
          pltpu.SemaphoreType.DMA,
          pltpu.SemaphoreType.DMA,
      ],
      compiler_params=pltpu.CompilerParams(use_tc_tiling_on_sc=False),
  )
  def edge_pass(g_hbm, src_hbm, dst_hbm, out_hbm,
                src_v, dst_v, rows_v, stage_v, acc_sh, gsem, ssem):
    c = lax.axis_index("c")
    s = lax.axis_index("s")
    wid = c * _NS + s

    @pl.loop(0, _RPW)
    def _(i):
      for d0 in range(dcols // 16):
        stage_v[i, pl.ds(d0 * 16, 16)] = jnp.zeros((16,), jnp.float32)

    rows = pl.ds(s * _RPW, _RPW)
    pltpu.sync_copy(stage_v, acc_sh.at[rows])
    pltpu.sync_copy(src_hbm.at[wid], src_v)
    pltpu.sync_copy(dst_hbm.at[wid], dst_v)
    plsc.subcore_barrier()

    @pl.loop(0, _K // _G)
    def _(grp):
      base = grp * _G
      gds = [
          pltpu.async_copy(g_hbm.at[src_v.at[base + b]], rows_v.at[b], gsem)
          for b in range(_G)
      ]
      for d in gds:
        d.wait()
      sds = [
          pltpu.async_copy(rows_v.at[b], acc_sh.at[dst_v.at[base + b]],
                           ssem, add=True)
          for b in range(_G)
      ]
      for d in sds:
        d.wait()

    plsc.subcore_barrier()
    pltpu.sync_copy(acc_sh.at[rows], out_hbm.at[c, rows])

  return edge_pass


_edge32 = _make_edge_pass(32)
_edge16 = _make_edge_pass(16)


@functools.partial(
    pl.kernel,
    out_type=jax.ShapeDtypeStruct((_NC, _NP, _DEGW), jnp.float32),
    mesh=_mesh,
    scratch_types=[
        pltpu.VMEM((_K, _B), jnp.int32),
        pltpu.VMEM((_B, _DEGW), jnp.float32),
        pltpu.VMEM((_RPW, _DEGW), jnp.float32),
        pltpu.VMEM_SHARED((_NP, _DEGW), jnp.float32),
        pltpu.SemaphoreType.DMA,
    ],
    compiler_params=pltpu.CompilerParams(use_tc_tiling_on_sc=False),
)
def _deg_pass(dst_hbm, out_hbm, dst_v, ones_v, stage_v, acc_sh, ssem):
  c = lax.axis_index("c")
  s = lax.axis_index("s")
  wid = c * _NS + s

  @pl.loop(0, _RPW)
  def _(i):
    stage_v[i, pl.ds(0, 16)] = jnp.zeros((16,), jnp.float32)

  @pl.loop(0, _B)
  def _(i):
    ones_v[i, pl.ds(0, 16)] = jnp.ones((16,), jnp.float32)

  rows = pl.ds(s * _RPW, _RPW)
  pltpu.sync_copy(stage_v, acc_sh.at[rows])
  pltpu.sync_copy(dst_hbm.at[wid], dst_v)
  plsc.subcore_barrier()

  @pl.loop(0, _K // _G)
  def _(grp):
    base = grp * _G
    sds = [
        pltpu.async_copy(ones_v, acc_sh.at[dst_v.at[base + b]],
                         ssem, add=True)
        for b in range(_G)
    ]
    for d in sds:
      d.wait()

  plsc.subcore_barrier()
  pltpu.sync_copy(acc_sh.at[rows], out_hbm.at[c, rows])


def _mm_body(x_ref, w_ref, o_ref):
  o_ref[...] = jnp.dot(x_ref[...], w_ref[...],
                       preferred_element_type=jnp.float32)


def _scale_body(degt_ref, hw_ref, g_ref, dis_ref):
  deg = degt_ref[0, :, 0:1] + degt_ref[1, :, 0:1] + 1.0
  row = lax.broadcasted_iota(jnp.int32, (_NP, 1), 0)
  dis = jnp.where(row < _N, lax.rsqrt(deg), 0.0)
  dis_ref[...] = dis
  g_ref[...] = hw_ref[...] * dis


def _mid_body(acc_ref, g_ref, dis_ref, b_ref, w_ref, o_ref):
  agg = (acc_ref[0] + acc_ref[1] + g_ref[...]) * dis_ref[...] + b_ref[...]
  h = jnp.maximum(agg, 0.0)
  o_ref[...] = jnp.dot(h, w_ref[...],
                       preferred_element_type=jnp.float32) * dis_ref[...]


def _final_body(acc_ref, g_ref, dis_ref, b_ref, w_ref, bo_ref, o_ref):
  agg = (acc_ref[0] + acc_ref[1] + g_ref[...]) * dis_ref[...] + b_ref[...]
  h = jnp.maximum(agg, 0.0)
  o_ref[...] = jnp.dot(h, w_ref[...],
                       preferred_element_type=jnp.float32) + bo_ref[...]


def _f32(*shape):
  return jax.ShapeDtypeStruct(shape, jnp.float32)


def kernel(x, edge_index, W1, b1, W2, b2, W_out, b_out):
  xp = jnp.pad(x, ((0, _NP - _N), (0, 0)))
  pad = _EP - _E
  fill = jnp.full((pad,), _N, jnp.int32)
  srcp = jnp.concatenate([edge_index[0], fill]).reshape(_NW, _K, _B)
  dstp = jnp.concatenate([edge_index[1], fill]).reshape(_NW, _K, _B)

  hw1 = pl.pallas_call(_mm_body, out_shape=_f32(_NP, 32))(xp, W1)
  degt = _deg_pass(dstp)
  g1, dis = pl.pallas_call(
      _scale_body, out_shape=(_f32(_NP, 32), _f32(_NP, 1)))(degt, hw1)
  acc1 = _edge32(g1, srcp, dstp)
  g2 = pl.pallas_call(_mid_body, out_shape=_f32(_NP, 16))(
      acc1, g1, dis, b1.reshape(1, -1), W2)
  acc2 = _edge16(g2, srcp, dstp)
  y = pl.pallas_call(_final_body, out_shape=_f32(_NP, 1))(
      acc2, g2, dis, b2.reshape(1, -1), W_out, b_out.reshape(1, 1))
  return y[:_N]


# speedup vs baseline: 1.5710x; 1.5710x over previous
import functools

import jax
import jax.numpy as jnp
from jax import lax
from jax.experimental import pallas as pl
from jax.experimental.pallas import tpu as pltpu
from jax.experimental.pallas import tpu_sc as plsc

_N = 10000
_E = 320000
_NC = 2
_NS = 16
_NW = _NC * _NS
_B = 128
_G = 8
_K = _G * (-(-_E // (_NW * _B * _G)))
_EP = _NW * _K * _B
_NP = 10112
_RPW = _NP // _NS
_DEGW = 16

_mesh = plsc.VectorSubcoreMesh(core_axis_name="c", subcore_axis_name="s")


def _make_edge_pass(dcols):
  @functools.partial(
      pl.kernel,
      out_type=jax.ShapeDtypeStruct((_NC, _NP, dcols), jnp.float32),
      mesh=_mesh,
      scratch_types=[
          pltpu.VMEM((_K, _B), jnp.int32),
          pltpu.VMEM((_K, _B), jnp.int32),
          pltpu.VMEM((_G, _B, dcols), jnp.float32),
          pltpu.VMEM((_RPW, dcols), jnp.float32),
          pltpu.VMEM_SHARED((_NP, dcols), jnp.float32),
          pltpu.VMEM_SHARED((_NP, dcols), jnp.float32),
          pltpu.SemaphoreType.DMA,
          pltpu.SemaphoreType.DMA,
      ],
      compiler_params=pltpu.CompilerParams(use_tc_tiling_on_sc=False),
  )
  def edge_pass(g_hbm, src_hbm, dst_hbm, out_hbm,
                src_v, dst_v, rows_v, stage_v, acc_sh, g_sh, gsem, ssem):
    c = lax.axis_index("c")
    s = lax.axis_index("s")
    wid = c * _NS + s

    @pl.loop(0, _RPW)
    def _(i):
      for d0 in range(dcols // 16):
        stage_v[i, pl.ds(d0 * 16, 16)] = jnp.zeros((16,), jnp.float32)

    rows = pl.ds(s * _RPW, _RPW)
    pltpu.sync_copy(stage_v, acc_sh.at[rows])
    pltpu.sync_copy(g_hbm.at[rows], g_sh.at[rows])
    pltpu.sync_copy(src_hbm.at[wid], src_v)
    pltpu.sync_copy(dst_hbm.at[wid], dst_v)
    plsc.subcore_barrier()

    @pl.loop(0, _K // _G)
    def _(grp):
      base = grp * _G
      gds = [
          pltpu.async_copy(g_sh.at[src_v.at[base + b]], rows_v.at[b], gsem)
          for b in range(_G)
      ]
      for d in gds:
        d.wait()
      sds = [
          pltpu.async_copy(rows_v.at[b], acc_sh.at[dst_v.at[base + b]],
                           ssem, add=True)
          for b in range(_G)
      ]
      for d in sds:
        d.wait()

    plsc.subcore_barrier()
    pltpu.sync_copy(acc_sh.at[rows], out_hbm.at[c, rows])

  return edge_pass


_edge32 = _make_edge_pass(32)
_edge16 = _make_edge_pass(16)


@functools.partial(
    pl.kernel,
    out_type=jax.ShapeDtypeStruct((_NC, _NP, _DEGW), jnp.float32),
    mesh=_mesh,
    scratch_types=[
        pltpu.VMEM((_K, _B), jnp.int32),
        pltpu.VMEM((_B, _DEGW), jnp.float32),
        pltpu.VMEM((_RPW, _DEGW), jnp.float32),
        pltpu.VMEM_SHARED((_NP, _DEGW), jnp.float32),
        pltpu.SemaphoreType.DMA,
    ],
    compiler_params=pltpu.CompilerParams(use_tc_tiling_on_sc=False),
)
def _deg_pass(dst_hbm, out_hbm, dst_v, ones_v, stage_v, acc_sh, ssem):
  c = lax.axis_index("c")
  s = lax.axis_index("s")
  wid = c * _NS + s

  @pl.loop(0, _RPW)
  def _(i):
    stage_v[i, pl.ds(0, 16)] = jnp.zeros((16,), jnp.float32)

  @pl.loop(0, _B)
  def _(i):
    ones_v[i, pl.ds(0, 16)] = jnp.ones((16,), jnp.float32)

  rows = pl.ds(s * _RPW, _RPW)
  pltpu.sync_copy(stage_v, acc_sh.at[rows])
  pltpu.sync_copy(dst_hbm.at[wid], dst_v)
  plsc.subcore_barrier()

  @pl.loop(0, _K // _G)
  def _(grp):
    base = grp * _G
    sds = [
        pltpu.async_copy(ones_v, acc_sh.at[dst_v.at[base + b]],
                         ssem, add=True)
        for b in range(_G)
    ]
    for d in sds:
      d.wait()

  plsc.subcore_barrier()
  pltpu.sync_copy(acc_sh.at[rows], out_hbm.at[c, rows])


def _mm_body(x_ref, w_ref, o_ref):
  o_ref[...] = jnp.dot(x_ref[...], w_ref[...],
                       preferred_element_type=jnp.float32)


def _scale_body(degt_ref, hw_ref, g_ref, dis_ref):
  deg = degt_ref[0, :, 0:1] + degt_ref[1, :, 0:1] + 1.0
  row = lax.broadcasted_iota(jnp.int32, (_NP, 1), 0)
  dis = jnp.where(row < _N, lax.rsqrt(deg), 0.0)
  dis_ref[...] = dis
  g_ref[...] = hw_ref[...] * dis


def _mid_body(acc_ref, g_ref, dis_ref, b_ref, w_ref, o_ref):
  agg = (acc_ref[0] + acc_ref[1] + g_ref[...]) * dis_ref[...] + b_ref[...]
  h = jnp.maximum(agg, 0.0)
  o_ref[...] = jnp.dot(h, w_ref[...],
                       preferred_element_type=jnp.float32) * dis_ref[...]


def _final_body(acc_ref, g_ref, dis_ref, b_ref, w_ref, bo_ref, o_ref):
  agg = (acc_ref[0] + acc_ref[1] + g_ref[...]) * dis_ref[...] + b_ref[...]
  h = jnp.maximum(agg, 0.0)
  o_ref[...] = jnp.dot(h, w_ref[...],
                       preferred_element_type=jnp.float32) + bo_ref[...]


def _f32(*shape):
  return jax.ShapeDtypeStruct(shape, jnp.float32)


def kernel(x, edge_index, W1, b1, W2, b2, W_out, b_out):
  xp = jnp.pad(x, ((0, _NP - _N), (0, 0)))
  pad = _EP - _E
  fill = jnp.full((pad,), _N, jnp.int32)
  srcp = jnp.concatenate([edge_index[0], fill]).reshape(_NW, _K, _B)
  dstp = jnp.concatenate([edge_index[1], fill]).reshape(_NW, _K, _B)

  hw1 = pl.pallas_call(_mm_body, out_shape=_f32(_NP, 32))(xp, W1)
  degt = _deg_pass(dstp)
  g1, dis = pl.pallas_call(
      _scale_body, out_shape=(_f32(_NP, 32), _f32(_NP, 1)))(degt, hw1)
  acc1 = _edge32(g1, srcp, dstp)
  g2 = pl.pallas_call(_mid_body, out_shape=_f32(_NP, 16))(
      acc1, g1, dis, b1.reshape(1, -1), W2)
  acc2 = _edge16(g2, srcp, dstp)
  y = pl.pallas_call(_final_body, out_shape=_f32(_NP, 1))(
      acc2, g2, dis, b2.reshape(1, -1), W_out, b_out.reshape(1, 1))
  return y[:_N]


# speedup vs baseline: 1.7086x; 1.0875x over previous
import functools

import jax
import jax.numpy as jnp
from jax import lax
from jax.experimental import pallas as pl
from jax.experimental.pallas import tpu as pltpu
from jax.experimental.pallas import tpu_sc as plsc

_N = 10000
_E = 320000
_NC = 2
_NS = 16
_NW = _NC * _NS
_B = 128
_G = 4
_K = _G * (-(-_E // (_NW * _B * _G)))
_EP = _NW * _K * _B
_NP = 10112
_RPW = _NP // _NS
_DEGW = 16

_mesh = plsc.VectorSubcoreMesh(core_axis_name="c", subcore_axis_name="s")


def _make_edge_pass(dcols):
  @functools.partial(
      pl.kernel,
      out_type=jax.ShapeDtypeStruct((_NC, _NP, dcols), jnp.float32),
      mesh=_mesh,
      scratch_types=[
          pltpu.VMEM((_K, _B), jnp.int32),
          pltpu.VMEM((_K, _B), jnp.int32),
          pltpu.VMEM((2, _G, _B, dcols), jnp.float32),
          pltpu.VMEM((_RPW, dcols), jnp.float32),
          pltpu.VMEM_SHARED((_NP, dcols), jnp.float32),
          pltpu.VMEM_SHARED((_NP, dcols), jnp.float32),
          pltpu.SemaphoreType.DMA((2,)),
          pltpu.SemaphoreType.DMA((2,)),
      ],
      compiler_params=pltpu.CompilerParams(use_tc_tiling_on_sc=False),
  )
  def edge_pass(g_hbm, src_hbm, dst_hbm, out_hbm,
                src_v, dst_v, rows_v, stage_v, acc_sh, g_sh, gsem, ssem):
    c = lax.axis_index("c")
    s = lax.axis_index("s")
    wid = c * _NS + s

    @pl.loop(0, _RPW)
    def _(i):
      for d0 in range(dcols // 16):
        stage_v[i, pl.ds(d0 * 16, 16)] = jnp.zeros((16,), jnp.float32)

    rows = pl.ds(s * _RPW, _RPW)
    pltpu.sync_copy(stage_v, acc_sh.at[rows])
    pltpu.sync_copy(g_hbm.at[rows], g_sh.at[rows])
    pltpu.sync_copy(src_hbm.at[wid], src_v)
    pltpu.sync_copy(dst_hbm.at[wid], dst_v)
    plsc.subcore_barrier()

    ngrp = _K // _G

    def gath(grp, p, b):
      return pltpu.make_async_copy(
          g_sh.at[src_v.at[grp * _G + b]], rows_v.at[p, b], gsem.at[p])

    def scat_start(grp, p, b):
      pltpu.async_copy(rows_v.at[p, b], acc_sh.at[dst_v.at[grp * _G + b]],
                       ssem.at[p], add=True)

    def scat_wait(grp, p, b):
      pltpu.make_async_copy(rows_v.at[p, b],
                            acc_sh.at[dst_v.at[grp * _G + b]],
                            ssem.at[p]).wait()

    for b in range(_G):
      gath(0, 0, b).start()

    @pl.loop(0, ngrp // 2)
    def _(m):
      g0 = m * 2
      g1 = g0 + 1

      @pl.when(m > 0)
      def _():
        for b in range(_G):
          scat_wait(g0 - 1, 1, b)

      for b in range(_G):
        gath(g1, 1, b).start()
      for b in range(_G):
        gath(g0, 0, b).wait()
      for b in range(_G):
        scat_start(g0, 0, b)
      for b in range(_G):
        scat_wait(g0, 0, b)

      @pl.when(g1 < ngrp - 1)
      def _():
        for b in range(_G):
          gath(g0 + 2, 0, b).start()

      for b in range(_G):
        gath(g1, 1, b).wait()
      for b in range(_G):
        scat_start(g1, 1, b)

    for b in range(_G):
      scat_wait(ngrp - 1, 1, b)

    plsc.subcore_barrier()
    pltpu.sync_copy(acc_sh.at[rows], out_hbm.at[c, rows])

  return edge_pass


_edge32 = _make_edge_pass(32)
_edge16 = _make_edge_pass(16)


@functools.partial(
    pl.kernel,
    out_type=jax.ShapeDtypeStruct((_NC, _NP, _DEGW), jnp.float32),
    mesh=_mesh,
    scratch_types=[
        pltpu.VMEM((_K, _B), jnp.int32),
        pltpu.VMEM((_B, _DEGW), jnp.float32),
        pltpu.VMEM((_RPW, _DEGW), jnp.float32),
        pltpu.VMEM_SHARED((_NP, _DEGW), jnp.float32),
        pltpu.SemaphoreType.DMA,
    ],
    compiler_params=pltpu.CompilerParams(use_tc_tiling_on_sc=False),
)
def _deg_pass(dst_hbm, out_hbm, dst_v, ones_v, stage_v, acc_sh, ssem):
  c = lax.axis_index("c")
  s = lax.axis_index("s")
  wid = c * _NS + s

  @pl.loop(0, _RPW)
  def _(i):
    stage_v[i, pl.ds(0, 16)] = jnp.zeros((16,), jnp.float32)

  @pl.loop(0, _B)
  def _(i):
    ones_v[i, pl.ds(0, 16)] = jnp.ones((16,), jnp.float32)

  rows = pl.ds(s * _RPW, _RPW)
  pltpu.sync_copy(stage_v, acc_sh.at[rows])
  pltpu.sync_copy(dst_hbm.at[wid], dst_v)
  plsc.subcore_barrier()

  ngrp = _K // _G

  def dscat(grp, b):
    return (ones_v, acc_sh.at[dst_v.at[grp * _G + b]], ssem)

  for b in range(_G):
    src, dst, sem = dscat(0, b)
    pltpu.async_copy(src, dst, sem, add=True)

  @pl.loop(1, ngrp)
  def _(grp):
    for b in range(_G):
      src, dst, sem = dscat(grp, b)
      pltpu.async_copy(src, dst, sem, add=True)
    for b in range(_G):
      src, dst, sem = dscat(grp - 1, b)
      pltpu.make_async_copy(src, dst, sem).wait()

  for b in range(_G):
    src, dst, sem = dscat(ngrp - 1, b)
    pltpu.make_async_copy(src, dst, sem).wait()

  plsc.subcore_barrier()
  pltpu.sync_copy(acc_sh.at[rows], out_hbm.at[c, rows])


def _mm_body(x_ref, w_ref, o_ref):
  hw = jnp.dot(x_ref[...], w_ref[...], preferred_element_type=jnp.float32)
  o_ref[...] = jnp.pad(hw, ((0, _NP - _N), (0, 0)))


def _scale_body(degt_ref, hw_ref, g_ref, dis_ref):
  deg = degt_ref[0, :, 0:1] + degt_ref[1, :, 0:1] + 1.0
  row = lax.broadcasted_iota(jnp.int32, (_NP, 1), 0)
  dis = jnp.where(row < _N, lax.rsqrt(deg), 0.0)
  dis_ref[...] = dis
  g_ref[...] = hw_ref[...] * dis


def _mid_body(acc_ref, g_ref, dis_ref, b_ref, w_ref, o_ref):
  agg = (acc_ref[0] + acc_ref[1] + g_ref[...]) * dis_ref[...] + b_ref[...]
  h = jnp.maximum(agg, 0.0)
  o_ref[...] = jnp.dot(h, w_ref[...],
                       preferred_element_type=jnp.float32) * dis_ref[...]


def _final_body(acc_ref, g_ref, dis_ref, b_ref, w_ref, bo_ref, o_ref):
  agg = (acc_ref[0] + acc_ref[1] + g_ref[...]) * dis_ref[...] + b_ref[...]
  h = jnp.maximum(agg, 0.0)
  y = jnp.dot(h, w_ref[...], preferred_element_type=jnp.float32) + bo_ref[...]
  o_ref[...] = y[:_N]


def _f32(*shape):
  return jax.ShapeDtypeStruct(shape, jnp.float32)


def kernel(x, edge_index, W1, b1, W2, b2, W_out, b_out):
  pad = _EP - _E
  fill = jnp.full((pad,), _N, jnp.int32)
  srcp = jnp.concatenate([edge_index[0], fill]).reshape(_NW, _K, _B)
  dstp = jnp.concatenate([edge_index[1], fill]).reshape(_NW, _K, _B)

  hw1 = pl.pallas_call(_mm_body, out_shape=_f32(_NP, 32))(x, W1)
  degt = _deg_pass(dstp)
  g1, dis = pl.pallas_call(
      _scale_body, out_shape=(_f32(_NP, 32), _f32(_NP, 1)))(degt, hw1)
  acc1 = _edge32(g1, srcp, dstp)
  g2 = pl.pallas_call(_mid_body, out_shape=_f32(_NP, 16))(
      acc1, g1, dis, b1.reshape(1, -1), W2)
  acc2 = _edge16(g2, srcp, dstp)
  y = pl.pallas_call(_final_body, out_shape=_f32(_N, 1))(
      acc2, g2, dis, b2.reshape(1, -1), W_out, b_out.reshape(1, 1))
  return y


# speedup vs baseline: 1.7960x; 1.0511x over previous
import functools

import jax
import jax.numpy as jnp
from jax import lax
from jax.experimental import pallas as pl
from jax.experimental.pallas import tpu as pltpu
from jax.experimental.pallas import tpu_sc as plsc

_N = 10000
_E = 320000
_NC = 2
_NS = 16
_NW = _NC * _NS
_B = 128
_G = 2
_K = _G * (-(-_E // (_NW * _B * _G)))
_EP = _NW * _K * _B
_NP = 10112
_RPW = _NP // _NS
_DEGW = 16

_mesh = plsc.VectorSubcoreMesh(core_axis_name="c", subcore_axis_name="s")
_params = pltpu.CompilerParams(use_tc_tiling_on_sc=False, needs_layout_passes=False)


def _rsqrt_nr(x):
  i = lax.bitcast_convert_type(x, jnp.int32)
  i = jnp.int32(0x5F3759DF) - lax.shift_right_arithmetic(i, 1)
  y = lax.bitcast_convert_type(i, jnp.float32)
  y = y * (1.5 - 0.5 * x * y * y)
  y = y * (1.5 - 0.5 * x * y * y)
  return y


@functools.partial(
    pl.kernel,
    out_type=jax.ShapeDtypeStruct((_NC, _NP), jnp.float32),
    mesh=_mesh,
    scratch_types=[
        pltpu.VMEM((_K, _B), jnp.int32),
        pltpu.VMEM((_B, _DEGW), jnp.float32),
        pltpu.VMEM((_RPW, _DEGW), jnp.float32),
        pltpu.VMEM((_RPW + 24,), jnp.float32),
        pltpu.VMEM_SHARED((_NP, _DEGW), jnp.float32),
        pltpu.SemaphoreType.DMA,
    ],
    compiler_params=_params,
)
def _deg_pass(dst_hbm, out_hbm, dst_v, ones_v, stage_v, dcol_v, acc_sh, ssem):
  c = lax.axis_index("c")
  s = lax.axis_index("s")
  wid = c * _NS + s

  @pl.loop(0, _RPW)
  def _(i):
    stage_v[i, pl.ds(0, 16)] = jnp.zeros((16,), jnp.float32)

  @pl.loop(0, _B)
  def _(i):
    ones_v[i, pl.ds(0, 16)] = jnp.ones((16,), jnp.float32)

  rows = pl.ds(s * _RPW, _RPW)
  pltpu.sync_copy(stage_v, acc_sh.at[rows])
  pltpu.sync_copy(dst_hbm.at[wid], dst_v)
  plsc.subcore_barrier()

  ngrp = _K // _G

  def dscat(grp, b):
    return (ones_v, acc_sh.at[dst_v.at[grp * _G + b]], ssem)

  for b in range(_G):
    src, dst, sem = dscat(0, b)
    pltpu.async_copy(src, dst, sem, add=True)

  @pl.loop(1, ngrp)
  def _(grp):
    for b in range(_G):
      src, dst, sem = dscat(grp, b)
      pltpu.async_copy(src, dst, sem, add=True)
    for b in range(_G):
      src, dst, sem = dscat(grp - 1, b)
      pltpu.make_async_copy(src, dst, sem).wait()

  for b in range(_G):
    src, dst, sem = dscat(ngrp - 1, b)
    pltpu.make_async_copy(src, dst, sem).wait()

  plsc.subcore_barrier()

  pltpu.sync_copy(acc_sh.at[rows], stage_v)
  zero16 = jnp.zeros((16,), jnp.int32)
  iota16 = lax.iota(jnp.int32, 16)

  @pl.loop(0, -(-_RPW // 16))
  def _(i):
    ridx = jnp.minimum(iota16 + i * 16, _RPW - 1)
    dcol_v[pl.ds(i * 16, 16)] = plsc.load_gather(stage_v, [ridx, zero16])

  pltpu.sync_copy(dcol_v.at[pl.ds(0, _RPW)], out_hbm.at[c, rows])


def _make_edge_pass(dcols, first_layer):
  outs = jax.ShapeDtypeStruct((_NC, _NP, dcols), jnp.float32)
  if first_layer:
    outs = (outs, jax.ShapeDtypeStruct((_NP,), jnp.float32))

  @functools.partial(
      pl.kernel,
      out_type=outs,
      mesh=_mesh,
      scratch_types=[
          pltpu.VMEM((_K, _B), jnp.int32),
          pltpu.VMEM((_K, _B), jnp.int32),
          pltpu.VMEM((2, _G, _B, dcols), jnp.float32),
          pltpu.VMEM((_RPW, dcols), jnp.float32),
          pltpu.VMEM((_RPW, dcols), jnp.float32),
          pltpu.VMEM((_RPW + 24,), jnp.float32),
          pltpu.VMEM((2, _RPW + 24), jnp.float32),
          pltpu.VMEM_SHARED((_NP, dcols), jnp.float32),
          pltpu.VMEM_SHARED((_NP, dcols), jnp.float32),
          pltpu.SemaphoreType.DMA((2,)),
          pltpu.SemaphoreType.DMA((2,)),
      ],
      compiler_params=_params,
  )
  def edge_pass(feat_hbm, aux_hbm, src_hbm, dst_hbm, *refs):
    if first_layer:
      (z_hbm, dis_hbm, src_v, dst_v, rows_v, g_v, z_v, dis_v, dc_v,
       acc_sh, g_sh, gsem, ssem) = refs
    else:
      (z_hbm, src_v, dst_v, rows_v, g_v, z_v, dis_v, dc_v,
       acc_sh, g_sh, gsem, ssem) = refs
    c = lax.axis_index("c")
    s = lax.axis_index("s")
    wid = c * _NS + s
    rows = pl.ds(s * _RPW, _RPW)

    @pl.loop(0, _RPW)
    def _(i):
      for d0 in range(dcols // 16):
        g_v[i, pl.ds(d0 * 16, 16)] = jnp.zeros((16,), jnp.float32)

    pltpu.sync_copy(g_v, acc_sh.at[rows])

    if first_layer:
      pltpu.sync_copy(aux_hbm.at[0, rows], dc_v.at[0, pl.ds(0, _RPW)])
      pltpu.sync_copy(aux_hbm.at[1, rows], dc_v.at[1, pl.ds(0, _RPW)])
      iota16 = lax.iota(jnp.int32, 16)
      base = s * _RPW

      @pl.loop(0, -(-_RPW // 16))
      def _(i):
        deg = dc_v[0, pl.ds(i * 16, 16)] + dc_v[1, pl.ds(i * 16, 16)] + 1.0
        ids = iota16 + (base + i * 16)
        dis_v[pl.ds(i * 16, 16)] = jnp.where(ids < _N, _rsqrt_nr(deg), 0.0)
    else:
      pltpu.sync_copy(aux_hbm.at[rows], dis_v.at[pl.ds(0, _RPW)])

    pltpu.sync_copy(feat_hbm.at[rows], g_v)

    @pl.loop(0, _RPW)
    def _(i):
      d = dis_v[pl.ds(i, 16)][0]
      for d0 in range(dcols // 16):
        sl = pl.ds(d0 * 16, 16)
        g_v[i, sl] = g_v[i, sl] * d

    pltpu.sync_copy(g_v, g_sh.at[rows])
    pltpu.sync_copy(src_hbm.at[wid], src_v)
    pltpu.sync_copy(dst_hbm.at[wid], dst_v)
    plsc.subcore_barrier()

    ngrp = _K // _G

    def gath(grp, p, b):
      return pltpu.make_async_copy(
          g_sh.at[src_v.at[grp * _G + b]], rows_v.at[p, b], gsem.at[p])

    def scat_start(grp, p, b):
      pltpu.async_copy(rows_v.at[p, b], acc_sh.at[dst_v.at[grp * _G + b]],
                       ssem.at[p], add=True)

    def scat_wait(grp, p, b):
      pltpu.make_async_copy(rows_v.at[p, b],
                            acc_sh.at[dst_v.at[grp * _G + b]],
                            ssem.at[p]).wait()

    for b in range(_G):
      gath(0, 0, b).start()

    @pl.loop(0, ngrp // 2)
    def _(m):
      g0 = m * 2
      g1 = g0 + 1

      @pl.when(m > 0)
      def _():
        for b in range(_G):
          scat_wait(g0 - 1, 1, b)

      for b in range(_G):
        gath(g1, 1, b).start()
      for b in range(_G):
        gath(g0, 0, b).wait()
      for b in range(_G):
        scat_start(g0, 0, b)
      for b in range(_G):
        scat_wait(g0, 0, b)

      @pl.when(g1 < ngrp - 1)
      def _():
        for b in range(_G):
          gath(g0 + 2, 0, b).start()

      for b in range(_G):
        gath(g1, 1, b).wait()
      for b in range(_G):
        scat_start(g1, 1, b)

    for b in range(_G):
      scat_wait(ngrp - 1, 1, b)

    plsc.subcore_barrier()

    pltpu.sync_copy(acc_sh.at[rows], z_v)
    gmul = jnp.where(c == 0, 1.0, 0.0)

    @pl.loop(0, _RPW)
    def _(i):
      d = dis_v[pl.ds(i, 16)][0]
      for d0 in range(dcols // 16):
        sl = pl.ds(d0 * 16, 16)
        z_v[i, sl] = (z_v[i, sl] + gmul * g_v[i, sl]) * d

    pltpu.sync_copy(z_v, z_hbm.at[c, rows])
    if first_layer:
      @pl.when(c == 0)
      def _():
        pltpu.sync_copy(dis_v.at[pl.ds(0, _RPW)], dis_hbm.at[rows])

  return edge_pass


_pass1 = _make_edge_pass(32, first_layer=True)
_pass2 = _make_edge_pass(16, first_layer=False)


def _mm_body(x_ref, w_ref, o_ref):
  hw = jnp.dot(x_ref[...], w_ref[...], preferred_element_type=jnp.float32)
  o_ref[...] = jnp.pad(hw, ((0, _NP - _N), (0, 0)))


def _mid_body(z_ref, b_ref, w_ref, o_ref):
  h = jnp.maximum(z_ref[0] + z_ref[1] + b_ref[...], 0.0)
  o_ref[...] = jnp.dot(h, w_ref[...], preferred_element_type=jnp.float32)


def _final_body(z_ref, b_ref, w_ref, bo_ref, o_ref):
  h = jnp.maximum(z_ref[0] + z_ref[1] + b_ref[...], 0.0)
  y = jnp.dot(h, w_ref[...], preferred_element_type=jnp.float32) + bo_ref[...]
  o_ref[...] = y[:_N]


def _f32(*shape):
  return jax.ShapeDtypeStruct(shape, jnp.float32)


def kernel(x, edge_index, W1, b1, W2, b2, W_out, b_out):
  pad = _EP - _E
  fill = jnp.full((pad,), _N, jnp.int32)
  srcp = jnp.concatenate([edge_index[0], fill]).reshape(_NW, _K, _B)
  dstp = jnp.concatenate([edge_index[1], fill]).reshape(_NW, _K, _B)

  hw1 = pl.pallas_call(_mm_body, out_shape=_f32(_NP, 32))(x, W1)
  degt = _deg_pass(dstp)
  z1, dis = _pass1(hw1, degt, srcp, dstp)
  p1 = pl.pallas_call(_mid_body, out_shape=_f32(_NP, 16))(
      z1, b1.reshape(1, -1), W2)
  z2 = _pass2(p1, dis, srcp, dstp)
  y = pl.pallas_call(_final_body, out_shape=_f32(_N, 1))(
      z2, b2.reshape(1, -1), W_out, b_out.reshape(1, 1))
  return y


# speedup vs baseline: 1.8490x; 1.0296x over previous
import functools

import jax
import jax.numpy as jnp
from jax import lax
from jax.experimental import pallas as pl
from jax.experimental.pallas import tpu as pltpu
from jax.experimental.pallas import tpu_sc as plsc

_N = 10000
_E = 320000
_NC = 2
_NS = 16
_NW = _NC * _NS
_B = 128
_G = 2
_K = _G * (-(-_E // (_NW * _B * _G)))
_EP = _NW * _K * _B
_NP = 10112
_RPW = _NP // _NS
_DEGW = 16

_mesh = plsc.VectorSubcoreMesh(core_axis_name="c", subcore_axis_name="s")
_params = pltpu.CompilerParams(use_tc_tiling_on_sc=False, needs_layout_passes=False)


def _rsqrt_nr(x):
  i = lax.bitcast_convert_type(x, jnp.int32)
  i = jnp.int32(0x5F3759DF) - lax.shift_right_arithmetic(i, 1)
  y = lax.bitcast_convert_type(i, jnp.float32)
  y = y * (1.5 - 0.5 * x * y * y)
  y = y * (1.5 - 0.5 * x * y * y)
  return y


@functools.partial(
    pl.kernel,
    out_type=jax.ShapeDtypeStruct((_NC, _NP), jnp.float32),
    mesh=_mesh,
    scratch_types=[
        pltpu.VMEM((_K, _B), jnp.int32),
        pltpu.VMEM((_B, _DEGW), jnp.float32),
        pltpu.VMEM((_RPW, _DEGW), jnp.float32),
        pltpu.VMEM((_RPW + 24,), jnp.float32),
        pltpu.VMEM_SHARED((_NP, _DEGW), jnp.float32),
        pltpu.SemaphoreType.DMA,
    ],
    compiler_params=_params,
)
def _deg_pass(dst_hbm, out_hbm, dst_v, ones_v, stage_v, dcol_v, acc_sh, ssem):
  c = lax.axis_index("c")
  s = lax.axis_index("s")
  wid = c * _NS + s

  @pl.loop(0, _RPW)
  def _(i):
    stage_v[i, pl.ds(0, 16)] = jnp.zeros((16,), jnp.float32)

  @pl.loop(0, _B)
  def _(i):
    ones_v[i, pl.ds(0, 16)] = jnp.ones((16,), jnp.float32)

  rows = pl.ds(s * _RPW, _RPW)
  pltpu.sync_copy(stage_v, acc_sh.at[rows])
  pltpu.sync_copy(dst_hbm.at[wid], dst_v)
  plsc.subcore_barrier()

  ngrp = _K // _G

  def dscat(grp, b):
    return (ones_v, acc_sh.at[dst_v.at[grp * _G + b]], ssem)

  for b in range(_G):
    src, dst, sem = dscat(0, b)
    pltpu.async_copy(src, dst, sem, add=True)

  @pl.loop(1, ngrp)
  def _(grp):
    for b in range(_G):
      src, dst, sem = dscat(grp, b)
      pltpu.async_copy(src, dst, sem, add=True)
    for b in range(_G):
      src, dst, sem = dscat(grp - 1, b)
      pltpu.make_async_copy(src, dst, sem).wait()

  for b in range(_G):
    src, dst, sem = dscat(ngrp - 1, b)
    pltpu.make_async_copy(src, dst, sem).wait()

  plsc.subcore_barrier()

  pltpu.sync_copy(acc_sh.at[rows], stage_v)
  zero16 = jnp.zeros((16,), jnp.int32)
  iota16 = lax.iota(jnp.int32, 16)

  @pl.loop(0, -(-_RPW // 16))
  def _(i):
    ridx = jnp.minimum(iota16 + i * 16, _RPW - 1)
    dcol_v[pl.ds(i * 16, 16)] = plsc.load_gather(stage_v, [ridx, zero16])

  pltpu.sync_copy(dcol_v.at[pl.ds(0, _RPW)], out_hbm.at[c, rows])


def _make_edge_pass(dcols, first_layer):
  outs = jax.ShapeDtypeStruct((_NC, _NP, dcols), jnp.float32)
  if first_layer:
    outs = (outs, jax.ShapeDtypeStruct((_NP,), jnp.float32))

  @functools.partial(
      pl.kernel,
      out_type=outs,
      mesh=_mesh,
      scratch_types=[
          pltpu.VMEM((_K, _B), jnp.int32),
          pltpu.VMEM((_K, _B), jnp.int32),
          pltpu.VMEM((2, _G, _B, dcols), jnp.float32),
          pltpu.VMEM((_RPW, dcols), jnp.float32),
          pltpu.VMEM((_RPW, dcols), jnp.float32),
          pltpu.VMEM((_RPW + 24,), jnp.float32),
          pltpu.VMEM((2, _RPW + 24), jnp.float32),
          pltpu.VMEM_SHARED((_NP, dcols), jnp.float32),
          pltpu.VMEM_SHARED((_NP, dcols), jnp.float32),
          pltpu.SemaphoreType.DMA((2,)),
          pltpu.SemaphoreType.DMA((2,)),
      ],
      compiler_params=_params,
  )
  def edge_pass(feat_hbm, aux_hbm, src_hbm, dst_hbm, *refs):
    if first_layer:
      (z_hbm, dis_hbm, src_v, dst_v, rows_v, g_v, z_v, dis_v, dc_v,
       acc_sh, g_sh, gsem, ssem) = refs
    else:
      (z_hbm, src_v, dst_v, rows_v, g_v, z_v, dis_v, dc_v,
       acc_sh, g_sh, gsem, ssem) = refs
    c = lax.axis_index("c")
    s = lax.axis_index("s")
    wid = c * _NS + s
    rows = pl.ds(s * _RPW, _RPW)

    @pl.loop(0, _RPW)
    def _(i):
      for d0 in range(dcols // 16):
        g_v[i, pl.ds(d0 * 16, 16)] = jnp.zeros((16,), jnp.float32)

    pltpu.sync_copy(g_v, acc_sh.at[rows])

    if first_layer:
      pltpu.sync_copy(aux_hbm.at[0, rows], dc_v.at[0, pl.ds(0, _RPW)])
      pltpu.sync_copy(aux_hbm.at[1, rows], dc_v.at[1, pl.ds(0, _RPW)])
      iota16 = lax.iota(jnp.int32, 16)
      base = s * _RPW

      @pl.loop(0, -(-_RPW // 16))
      def _(i):
        deg = dc_v[0, pl.ds(i * 16, 16)] + dc_v[1, pl.ds(i * 16, 16)] + 1.0
        ids = iota16 + (base + i * 16)
        dis_v[pl.ds(i * 16, 16)] = jnp.where(ids < _N, _rsqrt_nr(deg), 0.0)
    else:
      pltpu.sync_copy(aux_hbm.at[rows], dis_v.at[pl.ds(0, _RPW)])

    pltpu.sync_copy(feat_hbm.at[rows], g_v)

    @pl.loop(0, _RPW)
    def _(i):
      d = dis_v[pl.ds(i, 16)][0]
      for d0 in range(dcols // 16):
        sl = pl.ds(d0 * 16, 16)
        g_v[i, sl] = g_v[i, sl] * d

    pltpu.sync_copy(g_v, g_sh.at[rows])
    pltpu.sync_copy(src_hbm.at[wid], src_v)
    pltpu.sync_copy(dst_hbm.at[wid], dst_v)
    plsc.subcore_barrier()

    ngrp = _K // _G

    def gath(grp, p, b):
      return pltpu.make_async_copy(
          g_sh.at[src_v.at[grp * _G + b]], rows_v.at[p, b], gsem.at[p])

    def scat_start(grp, p, b):
      pltpu.async_copy(rows_v.at[p, b], acc_sh.at[dst_v.at[grp * _G + b]],
                       ssem.at[p], add=True)

    def scat_wait(grp, p, b):
      pltpu.make_async_copy(rows_v.at[p, b],
                            acc_sh.at[dst_v.at[grp * _G + b]],
                            ssem.at[p]).wait()

    for b in range(_G):
      gath(0, 0, b).start()

    @pl.loop(0, ngrp // 2)
    def _(m):
      g0 = m * 2
      g1 = g0 + 1

      @pl.when(m > 0)
      def _():
        for b in range(_G):
          scat_wait(g0 - 1, 1, b)

      for b in range(_G):
        gath(g1, 1, b).start()
      for b in range(_G):
        gath(g0, 0, b).wait()
      for b in range(_G):
        scat_start(g0, 0, b)
      for b in range(_G):
        scat_wait(g0, 0, b)

      @pl.when(g1 < ngrp - 1)
      def _():
        for b in range(_G):
          gath(g0 + 2, 0, b).start()

      for b in range(_G):
        gath(g1, 1, b).wait()
      for b in range(_G):
        scat_start(g1, 1, b)

    for b in range(_G):
      scat_wait(ngrp - 1, 1, b)

    plsc.subcore_barrier()

    pltpu.sync_copy(acc_sh.at[rows], z_v)
    gmul = jnp.where(c == 0, 1.0, 0.0)

    @pl.loop(0, _RPW)
    def _(i):
      d = dis_v[pl.ds(i, 16)][0]
      for d0 in range(dcols // 16):
        sl = pl.ds(d0 * 16, 16)
        z_v[i, sl] = (z_v[i, sl] + gmul * g_v[i, sl]) * d

    pltpu.sync_copy(z_v, z_hbm.at[c, rows])
    if first_layer:
      @pl.when(c == 0)
      def _():
        pltpu.sync_copy(dis_v.at[pl.ds(0, _RPW)], dis_hbm.at[rows])

  return edge_pass


_pass1 = _make_edge_pass(32, first_layer=True)
_pass2 = _make_edge_pass(16, first_layer=False)


@functools.partial(
    pl.kernel,
    out_type=jax.ShapeDtypeStruct((_NP,), jnp.float32),
    mesh=_mesh,
    scratch_types=[
        pltpu.VMEM((_RPW, 16), jnp.float32),
        pltpu.VMEM((_RPW, 16), jnp.float32),
        pltpu.VMEM((_RPW + 24,), jnp.float32),
        pltpu.VMEM((16,), jnp.float32),
        pltpu.VMEM((16,), jnp.float32),
        pltpu.VMEM((16,), jnp.float32),
    ],
    compiler_params=_params,
)
def _head_pass(z_hbm, b2_hbm, wo_hbm, bo_hbm, out_hbm,
               za_v, zb_v, y_v, b2_v, wo_v, bo_v):
  c = lax.axis_index("c")
  s = lax.axis_index("s")

  @pl.when(c == 0)
  def _():
    pltpu.sync_copy(b2_hbm, b2_v)
    pltpu.sync_copy(wo_hbm, wo_v)
    pltpu.sync_copy(bo_hbm, bo_v)
    rows = pl.ds(s * _RPW, _RPW)
    pltpu.sync_copy(z_hbm.at[0, rows], za_v)
    pltpu.sync_copy(z_hbm.at[1, rows], zb_v)
    b2r = b2_v[...]

    @pl.loop(0, _RPW)
    def _(i):
      sl = pl.ds(0, 16)
      za_v[i, sl] = jnp.maximum(za_v[i, sl] + zb_v[i, sl] + b2r, 0.0)

    wor = wo_v[...]
    bor = bo_v[...]
    iota16 = lax.iota(jnp.int32, 16)

    @pl.loop(0, -(-_RPW // 16))
    def _(blk):
      ridx = jnp.minimum(iota16 + blk * 16, _RPW - 1)
      acc = jnp.zeros((16,), jnp.float32) + bor[0]
      for j in range(16):
        col = plsc.load_gather(za_v, [ridx, jnp.full((16,), j, jnp.int32)])
        acc = acc + wor[j] * col
      y_v[pl.ds(blk * 16, 16)] = acc

    pltpu.sync_copy(y_v.at[pl.ds(0, _RPW)], out_hbm.at[rows])


def _mm_body(x_ref, w_ref, o_ref):
  hw = jnp.dot(x_ref[...], w_ref[...], preferred_element_type=jnp.float32)
  o_ref[...] = jnp.pad(hw, ((0, _NP - _N), (0, 0)))


def _mid_body(z_ref, b_ref, w_ref, o_ref):
  h = jnp.maximum(z_ref[0] + z_ref[1] + b_ref[...], 0.0)
  o_ref[...] = jnp.dot(h, w_ref[...], preferred_element_type=jnp.float32)


def _f32(*shape):
  return jax.ShapeDtypeStruct(shape, jnp.float32)


def kernel(x, edge_index, W1, b1, W2, b2, W_out, b_out):
  pad = _EP - _E
  fill = jnp.full((pad,), _N, jnp.int32)
  srcp = jnp.concatenate([edge_index[0], fill]).reshape(_NW, _K, _B)
  dstp = jnp.concatenate([edge_index[1], fill]).reshape(_NW, _K, _B)

  hw1 = pl.pallas_call(_mm_body, out_shape=_f32(_NP, 32))(x, W1)
  degt = _deg_pass(dstp)
  z1, dis = _pass1(hw1, degt, srcp, dstp)
  p1 = pl.pallas_call(_mid_body, out_shape=_f32(_NP, 16))(
      z1, b1.reshape(1, -1), W2)
  z2 = _pass2(p1, dis, srcp, dstp)
  y = _head_pass(z2, b2, W_out.reshape(-1), jnp.pad(b_out, (0, 15)))
  return y[:_N, None]


# speedup vs baseline: 1.8894x; 1.0218x over previous
import functools

import jax
import jax.numpy as jnp
from jax import lax
from jax.experimental import pallas as pl
from jax.experimental.pallas import tpu as pltpu
from jax.experimental.pallas import tpu_sc as plsc

_N = 10000
_E = 320000
_NC = 2
_NS = 16
_NW = _NC * _NS
_B = 128
_G = 2
_K = _G * (-(-_E // (_NW * _B * _G)))
_EP = _NW * _K * _B
_NP = 10112
_RPW = _NP // _NS
_DEGW = 16

_mesh = plsc.VectorSubcoreMesh(core_axis_name="c", subcore_axis_name="s")
_params = pltpu.CompilerParams(use_tc_tiling_on_sc=False, needs_layout_passes=False)


def _rsqrt_nr(x):
  i = lax.bitcast_convert_type(x, jnp.int32)
  i = jnp.int32(0x5F3759DF) - lax.shift_right_arithmetic(i, 1)
  y = lax.bitcast_convert_type(i, jnp.float32)
  y = y * (1.5 - 0.5 * x * y * y)
  y = y * (1.5 - 0.5 * x * y * y)
  return y


@functools.partial(
    pl.kernel,
    out_type=jax.ShapeDtypeStruct((_NC, _NP), jnp.float32),
    mesh=_mesh,
    scratch_types=[
        pltpu.VMEM((_K, _B), jnp.int32),
        pltpu.VMEM((_NP,), jnp.float32),
        pltpu.VMEM((_NS, _RPW + 24), jnp.float32),
        pltpu.VMEM((_RPW + 24,), jnp.float32),
        pltpu.VMEM_SHARED((_NS, _NP), jnp.float32),
        pltpu.SemaphoreType.DMA,
    ],
    compiler_params=_params,
)
def _deg_pass(dst_hbm, out_hbm, dst_v, hist_v, peer_v, dcol_v, slots_sh, sem):
  c = lax.axis_index("c")
  s = lax.axis_index("s")
  wid = c * _NS + s
  rows = pl.ds(s * _RPW, _RPW)

  @pl.loop(0, _NP // 16)
  def _(i):
    hist_v[pl.ds(i * 16, 16)] = jnp.zeros((16,), jnp.float32)

  pltpu.sync_copy(dst_hbm.at[wid], dst_v)
  ones16 = jnp.ones((16,), jnp.float32)

  @pl.loop(0, _K)
  def _(j):
    for m in range(_B // 16):
      plsc.addupdate_scatter(hist_v, [dst_v[j, pl.ds(m * 16, 16)]], ones16)

  pltpu.sync_copy(hist_v, slots_sh.at[s])
  plsc.subcore_barrier()

  for t in range(_NS):
    pltpu.async_copy(slots_sh.at[t, rows], peer_v.at[t, pl.ds(0, _RPW)], sem)
  for t in range(_NS):
    pltpu.make_async_copy(slots_sh.at[t, rows],
                          peer_v.at[t, pl.ds(0, _RPW)], sem).wait()

  @pl.loop(0, -(-_RPW // 16))
  def _(i):
    sl = pl.ds(i * 16, 16)
    acc = peer_v[0, sl]
    for t in range(1, _NS):
      acc = acc + peer_v[t, sl]
    dcol_v[sl] = acc

  pltpu.sync_copy(dcol_v.at[pl.ds(0, _RPW)], out_hbm.at[c, rows])


def _make_edge_pass(dcols, first_layer):
  outs = jax.ShapeDtypeStruct((_NC, _NP, dcols), jnp.float32)
  if first_layer:
    outs = (outs, jax.ShapeDtypeStruct((_NP,), jnp.float32))

  @functools.partial(
      pl.kernel,
      out_type=outs,
      mesh=_mesh,
      scratch_types=[
          pltpu.VMEM((_K, _B), jnp.int32),
          pltpu.VMEM((_K, _B), jnp.int32),
          pltpu.VMEM((2, _G, _B, dcols), jnp.float32),
          pltpu.VMEM((_RPW, dcols), jnp.float32),
          pltpu.VMEM((_RPW, dcols), jnp.float32),
          pltpu.VMEM((_RPW + 24,), jnp.float32),
          pltpu.VMEM((2, _RPW + 24), jnp.float32),
          pltpu.VMEM_SHARED((_NP, dcols), jnp.float32),
          pltpu.VMEM_SHARED((_NP, dcols), jnp.float32),
          pltpu.SemaphoreType.DMA((2,)),
          pltpu.SemaphoreType.DMA((2,)),
      ],
      compiler_params=_params,
  )
  def edge_pass(feat_hbm, aux_hbm, src_hbm, dst_hbm, *refs):
    if first_layer:
      (z_hbm, dis_hbm, src_v, dst_v, rows_v, g_v, z_v, dis_v, dc_v,
       acc_sh, g_sh, gsem, ssem) = refs
    else:
      (z_hbm, src_v, dst_v, rows_v, g_v, z_v, dis_v, dc_v,
       acc_sh, g_sh, gsem, ssem) = refs
    c = lax.axis_index("c")
    s = lax.axis_index("s")
    wid = c * _NS + s
    rows = pl.ds(s * _RPW, _RPW)

    @pl.loop(0, _RPW)
    def _(i):
      for d0 in range(dcols // 16):
        g_v[i, pl.ds(d0 * 16, 16)] = jnp.zeros((16,), jnp.float32)

    pltpu.sync_copy(g_v, acc_sh.at[rows])

    if first_layer:
      pltpu.sync_copy(aux_hbm.at[0, rows], dc_v.at[0, pl.ds(0, _RPW)])
      pltpu.sync_copy(aux_hbm.at[1, rows], dc_v.at[1, pl.ds(0, _RPW)])
      iota16 = lax.iota(jnp.int32, 16)
      base = s * _RPW

      @pl.loop(0, -(-_RPW // 16))
      def _(i):
        deg = dc_v[0, pl.ds(i * 16, 16)] + dc_v[1, pl.ds(i * 16, 16)] + 1.0
        ids = iota16 + (base + i * 16)
        dis_v[pl.ds(i * 16, 16)] = jnp.where(ids < _N, _rsqrt_nr(deg), 0.0)
    else:
      pltpu.sync_copy(aux_hbm.at[rows], dis_v.at[pl.ds(0, _RPW)])

    pltpu.sync_copy(feat_hbm.at[rows], g_v)

    @pl.loop(0, _RPW)
    def _(i):
      d = dis_v[pl.ds(i, 16)][0]
      for d0 in range(dcols // 16):
        sl = pl.ds(d0 * 16, 16)
        g_v[i, sl] = g_v[i, sl] * d

    pltpu.sync_copy(g_v, g_sh.at[rows])
    pltpu.sync_copy(src_hbm.at[wid], src_v)
    pltpu.sync_copy(dst_hbm.at[wid], dst_v)
    plsc.subcore_barrier()

    ngrp = _K // _G

    def gath(grp, p, b):
      return pltpu.make_async_copy(
          g_sh.at[src_v.at[grp * _G + b]], rows_v.at[p, b], gsem.at[p])

    def scat_start(grp, p, b):
      pltpu.async_copy(rows_v.at[p, b], acc_sh.at[dst_v.at[grp * _G + b]],
                       ssem.at[p], add=True)

    def scat_wait(grp, p, b):
      pltpu.make_async_copy(rows_v.at[p, b],
                            acc_sh.at[dst_v.at[grp * _G + b]],
                            ssem.at[p]).wait()

    for b in range(_G):
      gath(0, 0, b).start()

    @pl.loop(0, ngrp // 2)
    def _(m):
      g0 = m * 2
      g1 = g0 + 1

      @pl.when(m > 0)
      def _():
        for b in range(_G):
          scat_wait(g0 - 1, 1, b)

      for b in range(_G):
        gath(g1, 1, b).start()
      for b in range(_G):
        gath(g0, 0, b).wait()
      for b in range(_G):
        scat_start(g0, 0, b)
      for b in range(_G):
        scat_wait(g0, 0, b)

      @pl.when(g1 < ngrp - 1)
      def _():
        for b in range(_G):
          gath(g0 + 2, 0, b).start()

      for b in range(_G):
        gath(g1, 1, b).wait()
      for b in range(_G):
        scat_start(g1, 1, b)

    for b in range(_G):
      scat_wait(ngrp - 1, 1, b)

    plsc.subcore_barrier()

    pltpu.sync_copy(acc_sh.at[rows], z_v)
    gmul = jnp.where(c == 0, 1.0, 0.0)

    @pl.loop(0, _RPW)
    def _(i):
      d = dis_v[pl.ds(i, 16)][0]
      for d0 in range(dcols // 16):
        sl = pl.ds(d0 * 16, 16)
        z_v[i, sl] = (z_v[i, sl] + gmul * g_v[i, sl]) * d

    pltpu.sync_copy(z_v, z_hbm.at[c, rows])
    if first_layer:
      @pl.when(c == 0)
      def _():
        pltpu.sync_copy(dis_v.at[pl.ds(0, _RPW)], dis_hbm.at[rows])

  return edge_pass


_pass1 = _make_edge_pass(32, first_layer=True)
_pass2 = _make_edge_pass(16, first_layer=False)


@functools.partial(
    pl.kernel,
    out_type=jax.ShapeDtypeStruct((_NP,), jnp.float32),
    mesh=_mesh,
    scratch_types=[
        pltpu.VMEM((_RPW, 16), jnp.float32),
        pltpu.VMEM((_RPW, 16), jnp.float32),
        pltpu.VMEM((_RPW + 24,), jnp.float32),
        pltpu.VMEM((16,), jnp.float32),
        pltpu.VMEM((16,), jnp.float32),
        pltpu.VMEM((16,), jnp.float32),
    ],
    compiler_params=_params,
)
def _head_pass(z_hbm, b2_hbm, wo_hbm, bo_hbm, out_hbm,
               za_v, zb_v, y_v, b2_v, wo_v, bo_v):
  c = lax.axis_index("c")
  s = lax.axis_index("s")

  @pl.when(c == 0)
  def _():
    pltpu.sync_copy(b2_hbm, b2_v)
    pltpu.sync_copy(wo_hbm, wo_v)
    pltpu.sync_copy(bo_hbm, bo_v)
    rows = pl.ds(s * _RPW, _RPW)
    pltpu.sync_copy(z_hbm.at[0, rows], za_v)
    pltpu.sync_copy(z_hbm.at[1, rows], zb_v)
    b2r = b2_v[...]

    @pl.loop(0, _RPW)
    def _(i):
      sl = pl.ds(0, 16)
      za_v[i, sl] = jnp.maximum(za_v[i, sl] + zb_v[i, sl] + b2r, 0.0)

    wor = wo_v[...]
    bor = bo_v[...]
    iota16 = lax.iota(jnp.int32, 16)

    @pl.loop(0, -(-_RPW // 16))
    def _(blk):
      ridx = jnp.minimum(iota16 + blk * 16, _RPW - 1)
      acc = jnp.zeros((16,), jnp.float32) + bor[0]
      for j in range(16):
        col = plsc.load_gather(za_v, [ridx, jnp.full((16,), j, jnp.int32)])
        acc = acc + wor[j] * col
      y_v[pl.ds(blk * 16, 16)] = acc

    pltpu.sync_copy(y_v.at[pl.ds(0, _RPW)], out_hbm.at[rows])


def _mm_body(x_ref, w_ref, o_ref):
  hw = jnp.dot(x_ref[...], w_ref[...], preferred_element_type=jnp.float32)
  o_ref[...] = jnp.pad(hw, ((0, _NP - _N), (0, 0)))


def _mid_body(z_ref, b_ref, w_ref, o_ref):
  h = jnp.maximum(z_ref[0] + z_ref[1] + b_ref[...], 0.0)
  o_ref[...] = jnp.dot(h, w_ref[...], preferred_element_type=jnp.float32)


def _f32(*shape):
  return jax.ShapeDtypeStruct(shape, jnp.float32)


def kernel(x, edge_index, W1, b1, W2, b2, W_out, b_out):
  pad = _EP - _E
  fill = jnp.full((pad,), _N, jnp.int32)
  srcp = jnp.concatenate([edge_index[0], fill]).reshape(_NW, _K, _B)
  dstp = jnp.concatenate([edge_index[1], fill]).reshape(_NW, _K, _B)

  hw1 = pl.pallas_call(_mm_body, out_shape=_f32(_NP, 32))(x, W1)
  degt = _deg_pass(dstp)
  z1, dis = _pass1(hw1, degt, srcp, dstp)
  p1 = pl.pallas_call(_mid_body, out_shape=_f32(_NP, 16))(
      z1, b1.reshape(1, -1), W2)
  z2 = _pass2(p1, dis, srcp, dstp)
  y = _head_pass(z2, b2, W_out.reshape(-1), jnp.pad(b_out, (0, 15)))
  return y[:_N, None]
